# baseline (speedup 1.0000x reference)
import functools

import jax
import jax.numpy as jnp
from jax import lax
from jax.experimental import pallas as pl
from jax.experimental.pallas import tpu as pltpu
from jax.experimental.pallas import tpu_sc as plsc

VOCAB = 1000000
EMB_DIM = 64
BATCH = 4096
HIST = 50

_NC = 2
_NS = 16
_NW = _NC * _NS
_BB = BATCH // _NW
_HC = 5
_NCHUNK = HIST // _HC
_ROWS = _HC * _BB


def _gather_kernel(xp_hbm, table_hbm, out_hbm,
                   idx_v, rows0, rows1, dst_v, sem0, sem1):
    wid = lax.axis_index("s") * _NC + lax.axis_index("c")
    pltpu.sync_copy(xp_hbm.at[wid], idx_v)

    bufs = (rows0, rows1)
    sems = (sem0, sem1)

    def fire(c):
        pltpu.async_copy(
            table_hbm.at[idx_v.at[c]], bufs[c % 2], sems[c % 2])

    fire(0)
    for c in range(_NCHUNK):
        rows_v = bufs[c % 2]
        pltpu.make_async_copy(
            table_hbm.at[idx_v.at[c]], rows_v, sems[c % 2]).wait()
        if c + 1 < _NCHUNK:
            fire(c + 1)

        def pair(m, _):
            h = m // EMB_DIM
            e = m % EMB_DIM
            row0 = h * _BB
            col = jnp.full((16,), e, jnp.int32)
            for b16 in range(_BB // 16):
                row_ids = row0 + b16 * 16 + lax.iota(jnp.int32, 16)
                v = plsc.load_gather(rows_v, [row_ids, col])
                dst_v[h, e // 8, e % 8, pl.ds(b16 * 16, 16)] = v
            return _

        lax.fori_loop(0, _HC * EMB_DIM, pair, 0)
        pltpu.sync_copy(dst_v, out_hbm.at[pl.ds(c * _HC, _HC), :, wid])


@jax.jit
def _embed(xp, table):
    mesh = plsc.VectorSubcoreMesh(core_axis_name="c", subcore_axis_name="s")
    f = functools.partial(
        pl.kernel,
        mesh=mesh,
        out_type=jax.ShapeDtypeStruct(
            (HIST, EMB_DIM // 8, _NW, 8, _BB), jnp.float32),
        scratch_types=[
            pltpu.VMEM((_NCHUNK, _ROWS), jnp.int32),
            pltpu.VMEM((_ROWS, EMB_DIM), jnp.float32),
            pltpu.VMEM((_ROWS, EMB_DIM), jnp.float32),
            pltpu.VMEM((_HC, 8, 8, _BB), jnp.float32),
            pltpu.SemaphoreType.DMA,
            pltpu.SemaphoreType.DMA,
        ],
        compiler_params=pltpu.CompilerParams(
            use_tc_tiling_on_sc=False, needs_layout_passes=False),
    )(_gather_kernel)
    return f(xp, table)


def kernel(x, table):
    xp = (x.T.reshape(_NCHUNK, _HC, _NW, _BB)
          .transpose(2, 0, 1, 3).reshape(_NW, _NCHUNK, _ROWS))
    out5d = _embed(xp, table)
    return out5d.transpose(2, 4, 0, 1, 3).reshape(BATCH, HIST, EMB_DIM)
